# baseline (speedup 1.0000x reference)
import jax
import jax.numpy as jnp
from jax import lax
from jax.experimental import pallas as pl
from jax.experimental.pallas import tpu as pltpu
from jax.experimental.pallas import tpu_sc as plsc

B = 16384
EMB = 16
L = 16
NC = 2
NS = 16
NW = NC * NS
BPW = B // NW
NG = BPW // L

NU = 1000000
NI = 1000001
BLK = 128
NBLK = 7813
NPAD = NBLK * BLK
PB = 4
PHT = (NBLK + PB - 1) // PB
PH2 = 2 * PHT
QW = (PH2 + NW - 1) // NW
PAIRS = (QW + 1) // 2


def _transpose_body(ut_hbm, it_hbm, stagu, stagi,
                    ina, inb, touta, toutb,
                    sa0, sa1, sa2, sa3, sb0, sb1, sb2, sb3, soa, sob):
    sems_a = (sa0, sa1, sa2, sa3)
    sems_b = (sb0, sb1, sb2, sb3)
    wid = lax.axis_index("s") * NC + lax.axis_index("c")
    base = wid * QW
    lane = lax.iota(jnp.int32, L)
    t16 = lane >> 3
    s16 = lane & 7

    def fire_in(q2, bufs, sems):
        isu = q2 < PHT
        qt = jnp.where(isu, q2, q2 - PHT)
        for j in range(PB):
            c = jnp.minimum(qt * PB + j, NBLK - 1)
            off = pl.multiple_of(c * BLK, 128)

            @pl.when(isu)
            def _():
                pltpu.async_copy(ut_hbm.at[:, :, pl.ds(off, BLK)],
                                 bufs.at[j], sems[j])

            @pl.when(jnp.logical_not(isu))
            def _():
                pltpu.async_copy(it_hbm.at[:, :, pl.ds(off, BLK)],
                                 bufs.at[j], sems[j])

    def drain_in(bufs, sems):
        for j in range(PB):
            pltpu.make_async_copy(
                ut_hbm.at[:, :, pl.ds(0, BLK)], bufs.at[j], sems[j]).wait()

    def transpose_and_out(q2, bufs, tout, osem):
        isu = q2 < PHT
        qt = jnp.where(isu, q2, q2 - PHT)
        for j in range(PB):
            for l in range(BLK):
                v = plsc.load_gather(
                    bufs, [jnp.full((L,), j, jnp.int32), t16, s16,
                           jnp.full((L,), l, jnp.int32)])
                tout[pl.ds((j * BLK + l) * EMB, EMB)] = v
            c = jnp.minimum(qt * PB + j, NBLK - 1)
            roff = pl.multiple_of(c * (BLK * EMB), 128)

            @pl.when(isu)
            def _():
                pltpu.async_copy(tout.at[pl.ds(j * BLK * EMB, BLK * EMB)],
                                 stagu.at[pl.ds(roff, BLK * EMB)], osem)

            @pl.when(jnp.logical_not(isu))
            def _():
                pltpu.async_copy(tout.at[pl.ds(j * BLK * EMB, BLK * EMB)],
                                 stagi.at[pl.ds(roff, BLK * EMB)], osem)

    def drain_out(tout, osem):
        for j in range(PB):
            pltpu.make_async_copy(
                stagu.at[pl.ds(0, BLK * EMB)],
                tout.at[pl.ds(j * BLK * EMB, BLK * EMB)], osem).wait()

    fire_in(base + 0, ina, sems_a)
    fire_in(base + 1, inb, sems_b)
    drain_in(ina, sems_a)
    transpose_and_out(base + 0, ina, touta, soa)
    fire_in(base + 2, ina, sems_a)
    drain_in(inb, sems_b)
    transpose_and_out(base + 1, inb, toutb, sob)

    def pair(p, carry):
        qa = base + 2 * p
        fire_in(qa + 1, inb, sems_b)
        drain_out(touta, soa)
        drain_in(ina, sems_a)
        transpose_and_out(qa, ina, touta, soa)
        fire_in(qa + 2, ina, sems_a)
        drain_out(toutb, sob)
        drain_in(inb, sems_b)
        transpose_and_out(qa + 1, inb, toutb, sob)
        return carry

    lax.fori_loop(1, PAIRS, pair, 0)

    drain_out(touta, soa)
    drain_out(toutb, sob)
    drain_in(ina, sems_a)


def _gather_body(uidx_hbm, iidx_hbm, stagu, stagi, w_hbm, out_hbm,
                 uidxv, iidxv, urows, irows, outv, wv, sem_u, sem_i):
    wid = lax.axis_index("s") * NC + lax.axis_index("c")
    base = wid * BPW

    pltpu.sync_copy(uidx_hbm.at[pl.ds(base, BPW)], uidxv)
    pltpu.sync_copy(iidx_hbm.at[pl.ds(base, BPW)], iidxv)
    pltpu.sync_copy(w_hbm, wv)

    lane = lax.iota(jnp.int32, L)
    for g in range(NG):
        sl = pl.ds(g * L, L)
        uidxv[sl] = jnp.minimum(jnp.maximum(uidxv[sl], 0), NU - 1)
        iidxv[sl] = jnp.minimum(jnp.maximum(iidxv[sl], 0), NI - 1)

    cu = pltpu.async_copy(stagu.at[uidxv], urows, sem_u)
    ci = pltpu.async_copy(stagi.at[iidxv], irows, sem_i)
    cu.wait()
    ci.wait()

    wvec = wv[0]
    ws = [wvec[e] for e in range(EMB)]

    def compute(g, carry):
        rows = g * L + lane
        acc = jnp.zeros((L,), jnp.float32)
        for e in range(EMB):
            col = jnp.full((L,), e, jnp.int32)
            u = plsc.load_gather(urows, [rows, col])
            i = plsc.load_gather(irows, [rows, col])
            acc = acc + u * i * ws[e]
        outv[pl.ds(g * L, L)] = acc
        return carry

    lax.fori_loop(0, NG, compute, 0)

    pltpu.sync_copy(outv, out_hbm.at[pl.ds(base, BPW)])


def kernel(x, uid_table, iid_table, W):
    uidx = x[:, 0]
    iidx = x[:, 1]
    ut = uid_table.T.reshape(2, 8, NU)
    it = iid_table.T.reshape(2, 8, NI)
    mesh = plsc.VectorSubcoreMesh(
        core_axis_name="c", subcore_axis_name="s",
        num_cores=NC, num_subcores=NS)

    ktrans = pl.kernel(
        _transpose_body,
        out_type=[
            jax.ShapeDtypeStruct((NPAD * EMB,), jnp.float32),
            jax.ShapeDtypeStruct((NPAD * EMB,), jnp.float32),
        ],
        mesh=mesh,
        scratch_types=[
            pltpu.VMEM((PB, 2, 8, BLK), jnp.float32),
            pltpu.VMEM((PB, 2, 8, BLK), jnp.float32),
            pltpu.VMEM((PB * BLK * EMB,), jnp.float32),
            pltpu.VMEM((PB * BLK * EMB,), jnp.float32),
        ] + [pltpu.SemaphoreType.DMA] * 10,
        name="gmf_sc_transpose",
        compiler_params=pltpu.CompilerParams(
            needs_layout_passes=False,
            disable_bounds_checks=True,
            disable_semaphore_checks=True,
        ),
    )
    stagu, stagi = ktrans(ut, it)
    stagu = stagu.reshape(NPAD, EMB)
    stagi = stagi.reshape(NPAD, EMB)

    kgather = pl.kernel(
        _gather_body,
        out_type=jax.ShapeDtypeStruct((B,), jnp.float32),
        mesh=mesh,
        scratch_types=[
            pltpu.VMEM((BPW,), jnp.int32),
            pltpu.VMEM((BPW,), jnp.int32),
            pltpu.VMEM((BPW, EMB), jnp.float32),
            pltpu.VMEM((BPW, EMB), jnp.float32),
            pltpu.VMEM((BPW,), jnp.float32),
            pltpu.VMEM((1, EMB), jnp.float32),
            pltpu.SemaphoreType.DMA,
            pltpu.SemaphoreType.DMA,
        ],
        name="gmf_sc_gather",
        compiler_params=pltpu.CompilerParams(
            needs_layout_passes=False, use_tc_tiling_on_sc=False),
    )
    return kgather(uidx, iidx, stagu, stagi, W)
